# baseline (speedup 1.0000x reference)
import functools

import jax
import jax.numpy as jnp
from jax import lax
from jax.experimental import pallas as pl
from jax.experimental.pallas import tpu as pltpu, tpu_sc as plsc

N_NODES = 10000
N_EDGES = 160000
NODE_IN = 256
EDGE_IN = 16
HID = 512
N_LAYERS = 4
N_GRAPHS = 64

RN = 1536
N_RANGES = 8
N_PAD = RN * N_RANGES
KE = 64
EPAD = 2048
NSUB = 16
SUBR = HID // 128
ZR = 64
NBTOT = (N_EDGES + EPAD) // KE

_MESH = plsc.VectorSubcoreMesh(core_axis_name="c", subcore_axis_name="s")


def _sc_body(h_hbm, ea4_hbm, idxp_hbm, elo_hbm, aggr_hbm,
             srcv2, pb4, dlq2, idxb, elo_v, hbuf, ebuf4, zbuf, slab,
             sem_i, sem_g, sem_s):
    c = lax.axis_index("c")
    s = lax.axis_index("s")
    iota = lax.broadcasted_iota(jnp.int32, (16,), 0)

    pltpu.sync_copy(elo_hbm, elo_v)

    def zrow(i, carry):
        for u in range(8):
            zbuf[i, pl.ds(u * 16, 16)] = jnp.zeros((16,), jnp.float32)
        return carry

    lax.fori_loop(0, ZR, zrow, 0)

    rows_per_tile = RN * SUBR // NSUB

    def range_body(rr, carry0):
        r = c * (N_RANGES // 2) + rr
        e_lo = elo_v[pl.ds(r, 16)][0]
        e_hi = elo_v[pl.ds(r + 1, 16)][0]
        base_node = r * RN
        e_lo_al = (e_lo // KE) * KE
        nchunks = (e_hi - e_lo_al + (16 * KE - 1)) // (16 * KE)

        for j in range(rows_per_tile // ZR):
            pltpu.sync_copy(zbuf, slab.at[pl.ds(s * rows_per_tile + j * ZR, ZR)])
        plsc.subcore_barrier()

        blk0 = e_lo_al // KE + s
        pltpu.sync_copy(idxp_hbm.at[pl.ds(blk0, 1)], idxb)

        def chunk_body(j, carry):
            buf = j & 1
            base = e_lo_al + (j * 16 + s) * KE

            @pl.when(j > 0)
            def _():
                pltpu.make_async_copy(
                    idxp_hbm.at[pl.ds(blk0, 1)], idxb, sem_i).wait()

            for half in range(KE // 16):
                ev = base + half * 16 + iota
                valid = (ev >= e_lo) & (ev < e_hi)
                inv = jnp.where(valid, 0, 1)
                sv = idxb[0, 0, pl.ds(half * 16, 16)]
                srcv2[buf, pl.ds(half * 16, 16)] = jnp.where(valid, sv, 0)
                pv = jnp.where(valid, idxb[0, 1, pl.ds(half * 16, 16)], 0)
                dv = idxb[0, 2, pl.ds(half * 16, 16)]
                dvc = (jnp.where(valid, dv - base_node, RN)
                       + inv * (iota & 15))
                for q in range(SUBR):
                    pb4[buf, q, pl.ds(half * 16, 16)] = pv * SUBR + q
                    dlq2[buf, q, pl.ds(half * 16, 16)] = dvc * SUBR + q

            g_h = pltpu.async_copy(h_hbm.at[srcv2.at[buf]], hbuf, sem_g)

            @pl.when(j > 0)
            def _():
                for q in range(SUBR):
                    pltpu.make_async_copy(
                        ebuf4.at[q], slab.at[dlq2.at[1 - buf, q]],
                        sem_s).wait()

            g_e = [pltpu.async_copy(ea4_hbm.at[pb4.at[buf, q]],
                                    ebuf4.at[q], sem_g)
                   for q in range(SUBR)]

            @pl.when(j + 1 < nchunks)
            def _():
                blk = e_lo_al // KE + ((j + 1) * 16 + s)
                pltpu.async_copy(idxp_hbm.at[pl.ds(blk, 1)], idxb, sem_i)

            g_h.wait()
            for g in g_e:
                g.wait()

            def row_body(i, acc):
                for u in range(HID // 16):
                    v = hbuf[i, pl.ds(u * 16, 16)]
                    w = ebuf4[u // 8, i, pl.ds((u % 8) * 16, 16)]
                    ebuf4[u // 8, i, pl.ds((u % 8) * 16, 16)] = (
                        jnp.maximum(v + w, 0.0))
                return acc

            lax.fori_loop(0, KE, row_body, 0)

            for q in range(SUBR):
                pltpu.async_copy(ebuf4.at[q], slab.at[dlq2.at[buf, q]],
                                 sem_s, add=True)
            return carry

        lax.fori_loop(0, nchunks, chunk_body, 0)

        @pl.when(nchunks > 0)
        def _():
            lastbuf = (nchunks - 1) & 1
            for q in range(SUBR):
                pltpu.make_async_copy(
                    ebuf4.at[q], slab.at[dlq2.at[lastbuf, q]], sem_s).wait()

        plsc.subcore_barrier()
        pltpu.sync_copy(
            slab.at[pl.ds(s * rows_per_tile, rows_per_tile)],
            aggr_hbm.at[pl.ds(base_node * SUBR + s * rows_per_tile,
                              rows_per_tile)])
        plsc.subcore_barrier()
        return carry0

    lax.fori_loop(0, N_RANGES // 2, range_body, 0)


_sc_msg_pass = functools.partial(
    pl.kernel, mesh=_MESH,
    out_type=jax.ShapeDtypeStruct((N_PAD * SUBR, 128), jnp.float32),
    scratch_types=[
        pltpu.VMEM((2, KE), jnp.int32),
        pltpu.VMEM((2, SUBR, KE), jnp.int32),
        pltpu.VMEM((2, SUBR, KE), jnp.int32),
        pltpu.VMEM((1, 3, KE), jnp.int32),
        pltpu.VMEM((32,), jnp.int32),
        pltpu.VMEM((KE, HID), jnp.float32),
        pltpu.VMEM((SUBR, KE, 128), jnp.float32),
        pltpu.VMEM((ZR, 128), jnp.float32),
        pltpu.VMEM_SHARED(((RN + 16) * SUBR, 128), jnp.float32),
        pltpu.SemaphoreType.DMA,
        pltpu.SemaphoreType.DMA,
        pltpu.SemaphoreType.DMA,
    ],
)(_sc_body)


NB = 512
EB = 2048


def _embed_nodes_body(x_ref, wn_ref, bn_ref, out_ref):
    out_ref[...] = (
        lax.dot(x_ref[...], wn_ref[...], preferred_element_type=jnp.float32)
        + bn_ref[...]
    )


def _embed_nodes(x, Wn, bn):
    grid = (pl.cdiv(N_NODES, NB),)
    return pl.pallas_call(
        _embed_nodes_body,
        grid=grid,
        in_specs=[
            pl.BlockSpec((NB, NODE_IN), lambda i: (i, 0)),
            pl.BlockSpec((NODE_IN, HID), lambda i: (0, 0)),
            pl.BlockSpec((1, HID), lambda i: (0, 0)),
        ],
        out_specs=pl.BlockSpec((NB, HID), lambda i: (i, 0)),
        out_shape=jax.ShapeDtypeStruct((N_NODES, HID), jnp.float32),
    )(x, Wn, bn.reshape(1, HID))


def _embed_edges(edge_attr, We, be):
    grid = (pl.cdiv(N_EDGES, EB),)
    return pl.pallas_call(
        _embed_nodes_body,
        grid=grid,
        in_specs=[
            pl.BlockSpec((EB, EDGE_IN), lambda i: (i, 0)),
            pl.BlockSpec((EDGE_IN, HID), lambda i: (0, 0)),
            pl.BlockSpec((1, HID), lambda i: (0, 0)),
        ],
        out_specs=pl.BlockSpec((EB, HID), lambda i: (i, 0)),
        out_shape=jax.ShapeDtypeStruct((N_EDGES, HID), jnp.float32),
    )(edge_attr, We, be.reshape(1, HID))


def _mlp_body(h_ref, aggr_ref, w1_ref, b1_ref, w2_ref, b2_ref, out_ref):
    z = h_ref[...] + aggr_ref[...]
    t = jnp.maximum(
        lax.dot(z, w1_ref[...], preferred_element_type=jnp.float32)
        + b1_ref[...], 0.0)
    out_ref[...] = jnp.maximum(
        lax.dot(t, w2_ref[...], preferred_element_type=jnp.float32)
        + b2_ref[...], 0.0)


def _mlp(h, aggr, W1l, b1l, W2l, b2l):
    grid = (pl.cdiv(N_NODES, NB),)
    return pl.pallas_call(
        _mlp_body,
        grid=grid,
        in_specs=[
            pl.BlockSpec((NB, HID), lambda i: (i, 0)),
            pl.BlockSpec((NB, HID), lambda i: (i, 0)),
            pl.BlockSpec((HID, HID), lambda i: (0, 0)),
            pl.BlockSpec((1, HID), lambda i: (0, 0)),
            pl.BlockSpec((HID, HID), lambda i: (0, 0)),
            pl.BlockSpec((1, HID), lambda i: (0, 0)),
        ],
        out_specs=pl.BlockSpec((NB, HID), lambda i: (i, 0)),
        out_shape=jax.ShapeDtypeStruct((N_NODES, HID), jnp.float32),
    )(h, aggr, W1l, b1l.reshape(1, HID), W2l, b2l.reshape(1, HID))


POOL_BLK = 512


def _pool_body(batch_ref, h_ref, out_ref, cnt_ref):
    g = pl.program_id(0)
    nblk = pl.num_programs(0)
    row0 = g * POOL_BLK
    rows = lax.broadcasted_iota(jnp.int32, (POOL_BLK, 1), 0) + row0
    valid = rows < N_NODES
    b = batch_ref[0, 0].astype(jnp.int32).reshape(POOL_BLK, 1)
    gids = lax.broadcasted_iota(jnp.int32, (N_GRAPHS, POOL_BLK), 0)
    onehot = jnp.where((b.T == gids) & valid.T, 1.0, 0.0)

    @pl.when(g == 0)
    def _():
        out_ref[...] = jnp.zeros_like(out_ref)
        cnt_ref[...] = jnp.zeros_like(cnt_ref)

    out_ref[...] += lax.dot(onehot, h_ref[...],
                            preferred_element_type=jnp.float32)
    cnt_ref[...] += jnp.sum(onehot, axis=1, keepdims=True)

    @pl.when(g == nblk - 1)
    def _():
        out_ref[...] = out_ref[...] / jnp.maximum(cnt_ref[...], 1.0)


def _mean_pool(h, batch_i32):
    nblk = pl.cdiv(N_NODES, POOL_BLK)
    pad = nblk * POOL_BLK - N_NODES
    bpad = jnp.pad(batch_i32, (0, pad), constant_values=N_GRAPHS)
    bpad = bpad.reshape(nblk, 1, POOL_BLK)
    return pl.pallas_call(
        _pool_body,
        grid=(nblk,),
        in_specs=[
            pl.BlockSpec((1, 1, POOL_BLK), lambda g: (g, 0, 0)),
            pl.BlockSpec((POOL_BLK, HID), lambda g: (g, 0)),
        ],
        out_specs=pl.BlockSpec((N_GRAPHS, HID), lambda g: (0, 0)),
        out_shape=jax.ShapeDtypeStruct((N_GRAPHS, HID), jnp.float32),
        scratch_shapes=[pltpu.VMEM((N_GRAPHS, 1), jnp.float32)],
    )(bpad, h)


def kernel(x, edge_index, edge_attr, batch, Wn, bn, We, be, W1, b1, W2, b2):
    src = edge_index[0].astype(jnp.int32)
    dst = edge_index[1].astype(jnp.int32)

    perm = jnp.argsort(dst)
    dst_s = dst[perm]
    src_s = src[perm]
    elo = jnp.searchsorted(
        dst_s, jnp.arange(N_RANGES, dtype=jnp.int32) * RN).astype(jnp.int32)
    elo32 = jnp.concatenate(
        [elo, jnp.full((32 - N_RANGES,), N_EDGES, jnp.int32)])
    zpad = jnp.zeros((EPAD,), jnp.int32)
    src_p = jnp.concatenate([src_s, zpad]).reshape(NBTOT, KE)
    perm_p = jnp.concatenate([perm.astype(jnp.int32), zpad]).reshape(NBTOT, KE)
    dst_p = jnp.concatenate([dst_s, zpad]).reshape(NBTOT, KE)
    idxpack = jnp.stack([src_p, perm_p, dst_p], axis=1)

    h = _embed_nodes(x, Wn, bn)
    ea = _embed_edges(edge_attr, We, be)
    ea4 = ea.reshape(N_EDGES * SUBR, 128)

    for l in range(N_LAYERS):
        aggr = _sc_msg_pass(h, ea4, idxpack, elo32)
        aggr = aggr.reshape(N_PAD, HID)[:N_NODES]
        h = _mlp(h, aggr, W1[l], b1[l], W2[l], b2[l])

    return _mean_pool(h, batch.astype(jnp.int32))


# speedup vs baseline: 1.0512x; 1.0512x over previous
import functools

import jax
import jax.numpy as jnp
from jax import lax
from jax.experimental import pallas as pl
from jax.experimental.pallas import tpu as pltpu, tpu_sc as plsc

N_NODES = 10000
N_EDGES = 160000
NODE_IN = 256
EDGE_IN = 16
HID = 512
N_LAYERS = 4
N_GRAPHS = 64

RN = 1536
N_RANGES = 8
N_PAD = RN * N_RANGES
KE = 32
EPAD = 2048
NSUB = 16
SUBR = HID // 128
ZR = 64
NBTOT = (N_EDGES + EPAD) // KE

_MESH = plsc.VectorSubcoreMesh(core_axis_name="c", subcore_axis_name="s")


def _sc_body(h_hbm, ea_hbm, idxp_hbm, elo_hbm, aggr_hbm,
             srcv2, permv2, dlq2, idxb, elo_v, hbuf, ebuf, msgb, zbuf, slab,
             sem_i, sem_g, sem_s):
    c = lax.axis_index("c")
    s = lax.axis_index("s")
    iota = lax.broadcasted_iota(jnp.int32, (16,), 0)

    pltpu.sync_copy(elo_hbm, elo_v)

    def zrow(i, carry):
        for u in range(8):
            zbuf[i, pl.ds(u * 16, 16)] = jnp.zeros((16,), jnp.float32)
        return carry

    lax.fori_loop(0, ZR, zrow, 0)

    rows_per_tile = RN * SUBR // NSUB

    def range_body(rr, carry0):
        r = c * (N_RANGES // 2) + rr
        e_lo = elo_v[pl.ds(r, 16)][0]
        e_hi = elo_v[pl.ds(r + 1, 16)][0]
        base_node = r * RN
        e_lo_al = (e_lo // KE) * KE
        nchunks = (e_hi - e_lo_al + (16 * KE - 1)) // (16 * KE)

        for j in range(rows_per_tile // ZR):
            pltpu.sync_copy(zbuf, slab.at[pl.ds(s * rows_per_tile + j * ZR, ZR)])
        plsc.subcore_barrier()

        blk0 = e_lo_al // KE + s
        pltpu.sync_copy(idxp_hbm.at[pl.ds(blk0, 1)], idxb)

        def chunk_body(j, carry):
            buf = j & 1
            base = e_lo_al + (j * 16 + s) * KE

            @pl.when(j > 0)
            def _():
                pltpu.make_async_copy(
                    idxp_hbm.at[pl.ds(blk0, 1)], idxb, sem_i).wait()

            for half in range(KE // 16):
                ev = base + half * 16 + iota
                valid = (ev >= e_lo) & (ev < e_hi)
                inv = jnp.where(valid, 0, 1)
                sv = idxb[0, 0, pl.ds(half * 16, 16)]
                srcv2[buf, pl.ds(half * 16, 16)] = jnp.where(valid, sv, 0)
                pv = jnp.where(valid, idxb[0, 1, pl.ds(half * 16, 16)], 0)
                permv2[buf, pl.ds(half * 16, 16)] = pv
                dv = idxb[0, 2, pl.ds(half * 16, 16)]
                dvc = (jnp.where(valid, dv - base_node, RN)
                       + inv * (iota & 15))
                for q in range(SUBR):
                    dlq2[buf, q, pl.ds(half * 16, 16)] = dvc * SUBR + q

            g_h = pltpu.async_copy(h_hbm.at[srcv2.at[buf]], hbuf, sem_g)
            g_e = pltpu.async_copy(ea_hbm.at[permv2.at[buf]], ebuf, sem_g)

            @pl.when(j + 1 < nchunks)
            def _():
                blk = e_lo_al // KE + ((j + 1) * 16 + s)
                pltpu.async_copy(idxp_hbm.at[pl.ds(blk, 1)], idxb, sem_i)

            g_h.wait()
            g_e.wait()

            @pl.when(j > 0)
            def _():
                for q in range(SUBR):
                    pltpu.make_async_copy(
                        msgb.at[q], slab.at[dlq2.at[1 - buf, q]],
                        sem_s).wait()

            def row_body(i, acc):
                for u in range(HID // 16):
                    v = hbuf[i, pl.ds(u * 16, 16)]
                    w = ebuf[i, pl.ds(u * 16, 16)]
                    msgb[u // 8, i, pl.ds((u % 8) * 16, 16)] = (
                        jnp.maximum(v + w, 0.0))
                return acc

            lax.fori_loop(0, KE, row_body, 0)

            for q in range(SUBR):
                pltpu.async_copy(msgb.at[q], slab.at[dlq2.at[buf, q]],
                                 sem_s, add=True)
            return carry

        lax.fori_loop(0, nchunks, chunk_body, 0)

        @pl.when(nchunks > 0)
        def _():
            lastbuf = (nchunks - 1) & 1
            for q in range(SUBR):
                pltpu.make_async_copy(
                    msgb.at[q], slab.at[dlq2.at[lastbuf, q]], sem_s).wait()

        plsc.subcore_barrier()
        pltpu.sync_copy(
            slab.at[pl.ds(s * rows_per_tile, rows_per_tile)],
            aggr_hbm.at[pl.ds(base_node * SUBR + s * rows_per_tile,
                              rows_per_tile)])
        plsc.subcore_barrier()
        return carry0

    lax.fori_loop(0, N_RANGES // 2, range_body, 0)


_sc_msg_pass = functools.partial(
    pl.kernel, mesh=_MESH,
    out_type=jax.ShapeDtypeStruct((N_PAD * SUBR, 128), jnp.float32),
    scratch_types=[
        pltpu.VMEM((2, KE), jnp.int32),
        pltpu.VMEM((2, KE), jnp.int32),
        pltpu.VMEM((2, SUBR, KE), jnp.int32),
        pltpu.VMEM((1, 3, KE), jnp.int32),
        pltpu.VMEM((32,), jnp.int32),
        pltpu.VMEM((KE, HID), jnp.float32),
        pltpu.VMEM((KE, HID), jnp.float32),
        pltpu.VMEM((SUBR, KE, 128), jnp.float32),
        pltpu.VMEM((ZR, 128), jnp.float32),
        pltpu.VMEM_SHARED(((RN + 16) * SUBR, 128), jnp.float32),
        pltpu.SemaphoreType.DMA,
        pltpu.SemaphoreType.DMA,
        pltpu.SemaphoreType.DMA,
    ],
)(_sc_body)


NB = 512
EB = 2048


def _embed_nodes_body(x_ref, wn_ref, bn_ref, out_ref):
    out_ref[...] = (
        lax.dot(x_ref[...], wn_ref[...], preferred_element_type=jnp.float32)
        + bn_ref[...]
    )


def _embed_nodes(x, Wn, bn):
    grid = (pl.cdiv(N_NODES, NB),)
    return pl.pallas_call(
        _embed_nodes_body,
        grid=grid,
        in_specs=[
            pl.BlockSpec((NB, NODE_IN), lambda i: (i, 0)),
            pl.BlockSpec((NODE_IN, HID), lambda i: (0, 0)),
            pl.BlockSpec((1, HID), lambda i: (0, 0)),
        ],
        out_specs=pl.BlockSpec((NB, HID), lambda i: (i, 0)),
        out_shape=jax.ShapeDtypeStruct((N_NODES, HID), jnp.float32),
    )(x, Wn, bn.reshape(1, HID))


def _embed_edges(edge_attr, We, be):
    grid = (pl.cdiv(N_EDGES, EB),)
    return pl.pallas_call(
        _embed_nodes_body,
        grid=grid,
        in_specs=[
            pl.BlockSpec((EB, EDGE_IN), lambda i: (i, 0)),
            pl.BlockSpec((EDGE_IN, HID), lambda i: (0, 0)),
            pl.BlockSpec((1, HID), lambda i: (0, 0)),
        ],
        out_specs=pl.BlockSpec((EB, HID), lambda i: (i, 0)),
        out_shape=jax.ShapeDtypeStruct((N_EDGES, HID), jnp.float32),
    )(edge_attr, We, be.reshape(1, HID))


def _mlp_body(h_ref, aggr_ref, w1_ref, b1_ref, w2_ref, b2_ref, out_ref):
    z = h_ref[...] + aggr_ref[...]
    t = jnp.maximum(
        lax.dot(z, w1_ref[...], preferred_element_type=jnp.float32)
        + b1_ref[...], 0.0)
    out_ref[...] = jnp.maximum(
        lax.dot(t, w2_ref[...], preferred_element_type=jnp.float32)
        + b2_ref[...], 0.0)


def _mlp(h, aggr, W1l, b1l, W2l, b2l):
    grid = (pl.cdiv(N_NODES, NB),)
    return pl.pallas_call(
        _mlp_body,
        grid=grid,
        in_specs=[
            pl.BlockSpec((NB, HID), lambda i: (i, 0)),
            pl.BlockSpec((NB, HID), lambda i: (i, 0)),
            pl.BlockSpec((HID, HID), lambda i: (0, 0)),
            pl.BlockSpec((1, HID), lambda i: (0, 0)),
            pl.BlockSpec((HID, HID), lambda i: (0, 0)),
            pl.BlockSpec((1, HID), lambda i: (0, 0)),
        ],
        out_specs=pl.BlockSpec((NB, HID), lambda i: (i, 0)),
        out_shape=jax.ShapeDtypeStruct((N_NODES, HID), jnp.float32),
    )(h, aggr, W1l, b1l.reshape(1, HID), W2l, b2l.reshape(1, HID))


POOL_BLK = 512


def _pool_body(batch_ref, h_ref, out_ref, cnt_ref):
    g = pl.program_id(0)
    nblk = pl.num_programs(0)
    row0 = g * POOL_BLK
    rows = lax.broadcasted_iota(jnp.int32, (POOL_BLK, 1), 0) + row0
    valid = rows < N_NODES
    b = batch_ref[0, 0].astype(jnp.int32).reshape(POOL_BLK, 1)
    gids = lax.broadcasted_iota(jnp.int32, (N_GRAPHS, POOL_BLK), 0)
    onehot = jnp.where((b.T == gids) & valid.T, 1.0, 0.0)

    @pl.when(g == 0)
    def _():
        out_ref[...] = jnp.zeros_like(out_ref)
        cnt_ref[...] = jnp.zeros_like(cnt_ref)

    out_ref[...] += lax.dot(onehot, h_ref[...],
                            preferred_element_type=jnp.float32)
    cnt_ref[...] += jnp.sum(onehot, axis=1, keepdims=True)

    @pl.when(g == nblk - 1)
    def _():
        out_ref[...] = out_ref[...] / jnp.maximum(cnt_ref[...], 1.0)


def _mean_pool(h, batch_i32):
    nblk = pl.cdiv(N_NODES, POOL_BLK)
    pad = nblk * POOL_BLK - N_NODES
    bpad = jnp.pad(batch_i32, (0, pad), constant_values=N_GRAPHS)
    bpad = bpad.reshape(nblk, 1, POOL_BLK)
    return pl.pallas_call(
        _pool_body,
        grid=(nblk,),
        in_specs=[
            pl.BlockSpec((1, 1, POOL_BLK), lambda g: (g, 0, 0)),
            pl.BlockSpec((POOL_BLK, HID), lambda g: (g, 0)),
        ],
        out_specs=pl.BlockSpec((N_GRAPHS, HID), lambda g: (0, 0)),
        out_shape=jax.ShapeDtypeStruct((N_GRAPHS, HID), jnp.float32),
        scratch_shapes=[pltpu.VMEM((N_GRAPHS, 1), jnp.float32)],
    )(bpad, h)


def kernel(x, edge_index, edge_attr, batch, Wn, bn, We, be, W1, b1, W2, b2):
    src = edge_index[0].astype(jnp.int32)
    dst = edge_index[1].astype(jnp.int32)

    perm = jnp.argsort(dst)
    dst_s = dst[perm]
    src_s = src[perm]
    elo = jnp.searchsorted(
        dst_s, jnp.arange(N_RANGES, dtype=jnp.int32) * RN).astype(jnp.int32)
    elo32 = jnp.concatenate(
        [elo, jnp.full((32 - N_RANGES,), N_EDGES, jnp.int32)])
    zpad = jnp.zeros((EPAD,), jnp.int32)
    src_p = jnp.concatenate([src_s, zpad]).reshape(NBTOT, KE)
    perm_p = jnp.concatenate([perm.astype(jnp.int32), zpad]).reshape(NBTOT, KE)
    dst_p = jnp.concatenate([dst_s, zpad]).reshape(NBTOT, KE)
    idxpack = jnp.stack([src_p, perm_p, dst_p], axis=1)

    h = _embed_nodes(x, Wn, bn)
    ea = _embed_edges(edge_attr, We, be)

    for l in range(N_LAYERS):
        aggr = _sc_msg_pass(h, ea, idxpack, elo32)
        aggr = aggr.reshape(N_PAD, HID)[:N_NODES]
        h = _mlp(h, aggr, W1[l], b1[l], W2[l], b2[l])

    return _mean_pool(h, batch.astype(jnp.int32))


# speedup vs baseline: 1.1231x; 1.0683x over previous
import functools

import jax
import jax.numpy as jnp
from jax import lax
from jax.experimental import pallas as pl
from jax.experimental.pallas import tpu as pltpu, tpu_sc as plsc

N_NODES = 10000
N_EDGES = 160000
NODE_IN = 256
EDGE_IN = 16
HID = 512
N_LAYERS = 4
N_GRAPHS = 64

RN = 1792
N_RANGES = 6
N_PAD = RN * N_RANGES
KE = 32
EPAD = 1024
NSUB = 16

_MESH = plsc.VectorSubcoreMesh(core_axis_name="c", subcore_axis_name="s")


SUBR = HID // 128
ZR = 64


def _sc_body(h_hbm, ea_hbm, srcs_hbm, perms_hbm, dsts_hbm, elo_hbm, aggr_hbm,
             srcv, permv, dlv, dlq2, elo_v, hbuf, ebuf, msgb, zbuf, slab,
             sem1, sem2):
    c = lax.axis_index("c")
    s = lax.axis_index("s")
    iota = lax.broadcasted_iota(jnp.int32, (16,), 0)

    pltpu.sync_copy(elo_hbm, elo_v)

    def zrow(i, carry):
        for u in range(8):
            zbuf[i, pl.ds(u * 16, 16)] = jnp.zeros((16,), jnp.float32)
        return carry

    lax.fori_loop(0, ZR, zrow, 0)

    rows_per_tile = RN * SUBR // NSUB

    def range_body(rr, carry0):
        r = c * 3 + rr
        e_lo = elo_v[pl.ds(r, 16)][0]
        e_hi = elo_v[pl.ds(r + 1, 16)][0]
        base_node = r * RN
        e_lo_al = (e_lo // KE) * KE
        nchunks = (e_hi - e_lo_al + (16 * KE - 1)) // (16 * KE)

        for j in range(rows_per_tile // ZR):
            pltpu.sync_copy(zbuf, slab.at[pl.ds(s * rows_per_tile + j * ZR, ZR)])
        plsc.subcore_barrier()

        def chunk_body(j, carry):
            base = e_lo_al + (j * 16 + s) * KE
            ci1 = pltpu.async_copy(srcs_hbm.at[pl.ds(base, KE)], srcv, sem1)
            ci2 = pltpu.async_copy(perms_hbm.at[pl.ds(base, KE)], permv, sem1)
            ci3 = pltpu.async_copy(dsts_hbm.at[pl.ds(base, KE)], dlv, sem1)
            ci1.wait()
            ci2.wait()
            ci3.wait()
            for half in range(KE // 16):
                ev = base + half * 16 + iota
                valid = (ev >= e_lo) & (ev < e_hi)
                sv = srcv[pl.ds(half * 16, 16)]
                srcv[pl.ds(half * 16, 16)] = jnp.where(valid, sv, 0)
                pv = permv[pl.ds(half * 16, 16)]
                permv[pl.ds(half * 16, 16)] = jnp.where(valid, pv, 0)
                dv = dlv[pl.ds(half * 16, 16)]
                inv = jnp.where(valid, 0, 1)
                dvc = jnp.where(valid, dv - base_node, RN) + inv * (iota & 15)
                for q in range(SUBR):
                    dlq2[q, pl.ds(half * 16, 16)] = dvc * SUBR + q
            cp1 = pltpu.async_copy(h_hbm.at[srcv], hbuf, sem1)
            cp2 = pltpu.async_copy(ea_hbm.at[permv], ebuf, sem2)
            cp1.wait()
            cp2.wait()

            def row_body(i, acc):
                for u in range(HID // 16):
                    v = hbuf[i, pl.ds(u * 16, 16)]
                    w = ebuf[i, pl.ds(u * 16, 16)]
                    msgb[u // 8, i, pl.ds((u % 8) * 16, 16)] = (
                        jnp.maximum(v + w, 0.0))
                return acc

            lax.fori_loop(0, KE, row_body, 0)
            for q in range(SUBR):
                pltpu.sync_copy(msgb.at[q], slab.at[dlq2.at[q]], add=True)
            return carry

        lax.fori_loop(0, nchunks, chunk_body, 0)
        plsc.subcore_barrier()
        pltpu.sync_copy(
            slab.at[pl.ds(s * rows_per_tile, rows_per_tile)],
            aggr_hbm.at[pl.ds(base_node * SUBR + s * rows_per_tile,
                              rows_per_tile)])
        plsc.subcore_barrier()
        return carry0

    lax.fori_loop(0, 3, range_body, 0)


_sc_msg_pass = functools.partial(
    pl.kernel, mesh=_MESH,
    out_type=jax.ShapeDtypeStruct((N_PAD * SUBR, 128), jnp.float32),
    scratch_types=[
        pltpu.VMEM((KE,), jnp.int32),
        pltpu.VMEM((KE,), jnp.int32),
        pltpu.VMEM((KE,), jnp.int32),
        pltpu.VMEM((SUBR, KE), jnp.int32),
        pltpu.VMEM((32,), jnp.int32),
        pltpu.VMEM((KE, HID), jnp.float32),
        pltpu.VMEM((KE, HID), jnp.float32),
        pltpu.VMEM((SUBR, KE, 128), jnp.float32),
        pltpu.VMEM((ZR, 128), jnp.float32),
        pltpu.VMEM_SHARED(((RN + 16) * SUBR, 128), jnp.float32),
        pltpu.SemaphoreType.DMA,
        pltpu.SemaphoreType.DMA,
    ],
)(_sc_body)


NB = 512
EB = 2048


def _embed_nodes_body(x_ref, wn_ref, bn_ref, out_ref):
    out_ref[...] = (
        lax.dot(x_ref[...], wn_ref[...], preferred_element_type=jnp.float32)
        + bn_ref[...]
    )


def _embed_nodes(x, Wn, bn):
    grid = (pl.cdiv(N_NODES, NB),)
    return pl.pallas_call(
        _embed_nodes_body,
        grid=grid,
        in_specs=[
            pl.BlockSpec((NB, NODE_IN), lambda i: (i, 0)),
            pl.BlockSpec((NODE_IN, HID), lambda i: (0, 0)),
            pl.BlockSpec((1, HID), lambda i: (0, 0)),
        ],
        out_specs=pl.BlockSpec((NB, HID), lambda i: (i, 0)),
        out_shape=jax.ShapeDtypeStruct((N_NODES, HID), jnp.float32),
    )(x, Wn, bn.reshape(1, HID))


def _embed_edges(edge_attr, We, be):
    grid = (pl.cdiv(N_EDGES, EB),)
    return pl.pallas_call(
        _embed_nodes_body,
        grid=grid,
        in_specs=[
            pl.BlockSpec((EB, EDGE_IN), lambda i: (i, 0)),
            pl.BlockSpec((EDGE_IN, HID), lambda i: (0, 0)),
            pl.BlockSpec((1, HID), lambda i: (0, 0)),
        ],
        out_specs=pl.BlockSpec((EB, HID), lambda i: (i, 0)),
        out_shape=jax.ShapeDtypeStruct((N_EDGES, HID), jnp.float32),
    )(edge_attr, We, be.reshape(1, HID))


def _mlp_body(h_ref, aggr_ref, w1_ref, b1_ref, w2_ref, b2_ref, out_ref):
    z = h_ref[...] + aggr_ref[...]
    t = jnp.maximum(
        lax.dot(z, w1_ref[...], preferred_element_type=jnp.float32)
        + b1_ref[...], 0.0)
    out_ref[...] = jnp.maximum(
        lax.dot(t, w2_ref[...], preferred_element_type=jnp.float32)
        + b2_ref[...], 0.0)


def _mlp(h, aggr, W1l, b1l, W2l, b2l):
    grid = (pl.cdiv(N_NODES, NB),)
    return pl.pallas_call(
        _mlp_body,
        grid=grid,
        in_specs=[
            pl.BlockSpec((NB, HID), lambda i: (i, 0)),
            pl.BlockSpec((NB, HID), lambda i: (i, 0)),
            pl.BlockSpec((HID, HID), lambda i: (0, 0)),
            pl.BlockSpec((1, HID), lambda i: (0, 0)),
            pl.BlockSpec((HID, HID), lambda i: (0, 0)),
            pl.BlockSpec((1, HID), lambda i: (0, 0)),
        ],
        out_specs=pl.BlockSpec((NB, HID), lambda i: (i, 0)),
        out_shape=jax.ShapeDtypeStruct((N_NODES, HID), jnp.float32),
    )(h, aggr, W1l, b1l.reshape(1, HID), W2l, b2l.reshape(1, HID))


POOL_BLK = 512


def _pool_body(batch_ref, h_ref, out_ref, cnt_ref):
    g = pl.program_id(0)
    nblk = pl.num_programs(0)
    row0 = g * POOL_BLK
    rows = lax.broadcasted_iota(jnp.int32, (POOL_BLK, 1), 0) + row0
    valid = rows < N_NODES
    b = batch_ref[0, 0].astype(jnp.int32).reshape(POOL_BLK, 1)
    gids = lax.broadcasted_iota(jnp.int32, (N_GRAPHS, POOL_BLK), 0)
    onehot = jnp.where((b.T == gids) & valid.T, 1.0, 0.0)

    @pl.when(g == 0)
    def _():
        out_ref[...] = jnp.zeros_like(out_ref)
        cnt_ref[...] = jnp.zeros_like(cnt_ref)

    out_ref[...] += lax.dot(onehot, h_ref[...],
                            preferred_element_type=jnp.float32)
    cnt_ref[...] += jnp.sum(onehot, axis=1, keepdims=True)

    @pl.when(g == nblk - 1)
    def _():
        out_ref[...] = out_ref[...] / jnp.maximum(cnt_ref[...], 1.0)


def _mean_pool(h, batch_i32):
    nblk = pl.cdiv(N_NODES, POOL_BLK)
    pad = nblk * POOL_BLK - N_NODES
    bpad = jnp.pad(batch_i32, (0, pad), constant_values=N_GRAPHS)
    bpad = bpad.reshape(nblk, 1, POOL_BLK)
    return pl.pallas_call(
        _pool_body,
        grid=(nblk,),
        in_specs=[
            pl.BlockSpec((1, 1, POOL_BLK), lambda g: (g, 0, 0)),
            pl.BlockSpec((POOL_BLK, HID), lambda g: (g, 0)),
        ],
        out_specs=pl.BlockSpec((N_GRAPHS, HID), lambda g: (0, 0)),
        out_shape=jax.ShapeDtypeStruct((N_GRAPHS, HID), jnp.float32),
        scratch_shapes=[pltpu.VMEM((N_GRAPHS, 1), jnp.float32)],
    )(bpad, h)


def kernel(x, edge_index, edge_attr, batch, Wn, bn, We, be, W1, b1, W2, b2):
    src = edge_index[0].astype(jnp.int32)
    dst = edge_index[1].astype(jnp.int32)

    perm = jnp.argsort(dst)
    dst_s = dst[perm]
    src_s = src[perm]
    elo = jnp.searchsorted(
        dst_s, jnp.arange(N_RANGES, dtype=jnp.int32) * RN).astype(jnp.int32)
    elo16 = jnp.concatenate(
        [elo, jnp.full((32 - N_RANGES,), N_EDGES, jnp.int32)])
    zpad = jnp.zeros((EPAD,), jnp.int32)
    src_p = jnp.concatenate([src_s, zpad])
    perm_p = jnp.concatenate([perm.astype(jnp.int32), zpad])
    dst_p = jnp.concatenate([dst_s, zpad])

    h = _embed_nodes(x, Wn, bn)
    ea = _embed_edges(edge_attr, We, be)

    for l in range(N_LAYERS):
        aggr = _sc_msg_pass(h, ea, src_p, perm_p, dst_p, elo16)
        aggr = aggr.reshape(N_PAD, HID)[:N_NODES]
        h = _mlp(h, aggr, W1[l], b1[l], W2[l], b2[l])

    return _mean_pool(h, batch.astype(jnp.int32))
